# baseline (speedup 1.0000x reference)
import jax
import jax.numpy as jnp
from jax import lax
from jax.experimental import pallas as pl
from jax.experimental.pallas import tpu as pltpu
from jax.experimental.pallas import tpu_sc as plsc

B = 16384
L = 50
D = 64
V = 1000000
NG = 2 * B
NW = 32
GPW = NG // NW
IB = 64
NB = GPW // IB
NBUF = 4
LANES = 16
NV = D // LANES
INV_L = 1.0 / L
LG = 56

RW = 2048
RH = RW // 2
NRB = (V + RW - 1) // RW
VP = NRB * RW


def _relayout_body(x_ref, o_ref):
    x = x_ref[...]
    o_ref[...] = jnp.concatenate([x[:, 0:RH].T, x[:, RH:RW].T], axis=1)


@jax.jit
def _relayout(tt):
    return pl.pallas_call(
        _relayout_body,
        grid=(NRB,),
        in_specs=[pl.BlockSpec((D, RW), lambda j: (0, j))],
        out_specs=pl.BlockSpec((RH, 2 * D), lambda j: (j, 0)),
        out_shape=jax.ShapeDtypeStruct((VP // 2, 2 * D), jnp.float32),
    )(tt)


def _pool_body(idx_hbm, table_hbm, out_hbm,
               idx_v, r0, r1, r2, r3, outblk, s0, s1, s2, s3, sob):
    wid = lax.axis_index("s") * 2 + lax.axis_index("c")
    base = wid * GPW
    rows = (r0, r1, r2, r3)
    sems = (s0, s1, s2, s3)

    def remap(r, _):
        vs = [idx_v[r, pl.ds(c, LANES)] for c in (0, 16, 32, 40)]
        outs = []
        for v in vs:
            off = v & (RW - 1)
            outs.append(v + off - jnp.where(off < RH, 0, RW - 1))
        for c, o in zip((0, 16, 32, 40), outs):
            idx_v[r, pl.ds(c, LANES)] = o
        return 0

    def accumulate(buf, g):
        def rbody(r, accs):
            return tuple(
                accs[j] + buf[r, pl.ds(j * LANES, LANES)] for j in range(NV)
            )
        init = tuple(buf[0, pl.ds(j * LANES, LANES)] for j in range(NV))
        accs = lax.fori_loop(1, L, rbody, init, unroll=5)
        for j in range(NV):
            outblk[g, pl.ds(j * LANES, LANES)] = accs[j] * INV_L

    def block_body(blk):
        @pl.when(blk > 0)
        def _():
            pltpu.make_async_copy(
                outblk, out_hbm.at[pl.ds(base + (blk - 1) * IB, IB)], sob).wait()

        row0 = base + blk * IB
        pltpu.sync_copy(idx_hbm.at[pl.ds(row0, IB)], idx_v)
        lax.fori_loop(0, IB, remap, 0, unroll=2)
        for s in range(NBUF):
            pltpu.async_copy(
                table_hbm.at[idx_v.at[s]], rows[s], sems[s])

        def quad(p):
            for s in range(NBUF):
                g = NBUF * p + s
                pltpu.make_async_copy(
                    table_hbm.at[idx_v.at[g]], rows[s], sems[s]).wait()
                accumulate(rows[s], g)

                @pl.when(g + NBUF < IB)
                def _():
                    pltpu.async_copy(
                        table_hbm.at[idx_v.at[g + NBUF]], rows[s], sems[s])

        pl.loop(0, IB // NBUF)(quad)
        pltpu.async_copy(outblk, out_hbm.at[pl.ds(row0, IB)], sob)

    pl.loop(0, NB)(block_body)
    pltpu.make_async_copy(
        outblk, out_hbm.at[pl.ds(base + (NB - 1) * IB, IB)], sob).wait()


@jax.jit
def _pooled_lookup(idx, table_lin):
    mesh = plsc.VectorSubcoreMesh(core_axis_name="c", subcore_axis_name="s")
    return pl.kernel(
        _pool_body,
        out_type=jax.ShapeDtypeStruct((NG, D), jnp.float32),
        mesh=mesh,
        scratch_types=[
            pltpu.VMEM((IB, LG), jnp.int32),
            pltpu.VMEM((LG, D), jnp.float32),
            pltpu.VMEM((LG, D), jnp.float32),
            pltpu.VMEM((LG, D), jnp.float32),
            pltpu.VMEM((LG, D), jnp.float32),
            pltpu.VMEM((IB, D), jnp.float32),
            pltpu.SemaphoreType.DMA,
            pltpu.SemaphoreType.DMA,
            pltpu.SemaphoreType.DMA,
            pltpu.SemaphoreType.DMA,
            pltpu.SemaphoreType.DMA,
        ],
        compiler_params=pltpu.CompilerParams(use_tc_tiling_on_sc=False),
    )(idx, table_lin)


def _mm_body(x_ref, w_ref, b_ref, o_ref):
    o_ref[...] = (
        jnp.dot(x_ref[...], w_ref[...], preferred_element_type=jnp.float32)
        + b_ref[...]
    )


@jax.jit
def _project(pooled, W, b):
    blk = 4096
    return pl.pallas_call(
        _mm_body,
        grid=(NG // blk,),
        in_specs=[
            pl.BlockSpec((blk, D), lambda i: (i, 0)),
            pl.BlockSpec((D, D), lambda i: (0, 0)),
            pl.BlockSpec((1, D), lambda i: (0, 0)),
        ],
        out_specs=pl.BlockSpec((blk, D), lambda i: (i, 0)),
        out_shape=jax.ShapeDtypeStruct((NG, D), jnp.float32),
    )(pooled, W, b.reshape(1, D))


def kernel(query, candidate, table, W, b):
    idx = jnp.concatenate([query, candidate], axis=0).astype(jnp.int32)
    idx = jnp.pad(idx, ((0, 0), (0, LG - L)))
    table2 = _relayout(table.T)
    table_lin = table2.reshape(VP, D)
    pooled = _pooled_lookup(idx, table_lin)
    out = _project(pooled, W, b)
    return (out[:B], out[B:])


# speedup vs baseline: 5.3873x; 5.3873x over previous
import jax
import jax.numpy as jnp
from jax import lax
from jax.experimental import pallas as pl
from jax.experimental.pallas import tpu as pltpu
from jax.experimental.pallas import tpu_sc as plsc

B = 16384
L = 50
D = 64
V = 1000000
NG = 2 * B
NW = 32
GPW = NG // NW
IB = 64
NB = GPW // IB
NBUF = 4
LANES = 16
NV = D // LANES
INV_L = 1.0 / L
LG = 56

RW = 2048
RH = RW // 2
NRB = (V + RW - 1) // RW
VP = NRB * RW


def _relayout_body(x_ref, o_ref):
    x = x_ref[...]
    eye = jnp.eye(D, dtype=jnp.float32)
    dn = (((0,), (0,)), ((), ()))
    lo = lax.dot_general(x[:, 0:RH], eye, dn, preferred_element_type=jnp.float32)
    hi = lax.dot_general(x[:, RH:RW], eye, dn, preferred_element_type=jnp.float32)
    o_ref[...] = jnp.concatenate([lo, hi], axis=1)


@jax.jit
def _relayout(tt):
    return pl.pallas_call(
        _relayout_body,
        grid=(NRB,),
        in_specs=[pl.BlockSpec((D, RW), lambda j: (0, j))],
        out_specs=pl.BlockSpec((RH, 2 * D), lambda j: (j, 0)),
        out_shape=jax.ShapeDtypeStruct((VP // 2, 2 * D), jnp.float32),
    )(tt)


def _pool_body(idx_hbm, table_hbm, out_hbm,
               idx_v, r0, r1, r2, r3, outblk, s0, s1, s2, s3, sob):
    wid = lax.axis_index("s") * 2 + lax.axis_index("c")
    base = wid * GPW
    rows = (r0, r1, r2, r3)
    sems = (s0, s1, s2, s3)

    def remap(r, _):
        vs = [idx_v[r, pl.ds(c, LANES)] for c in (0, 16, 32, 40)]
        outs = []
        for v in vs:
            off = v & (RW - 1)
            outs.append(v + off - jnp.where(off < RH, 0, RW - 1))
        for c, o in zip((0, 16, 32, 40), outs):
            idx_v[r, pl.ds(c, LANES)] = o
        return 0

    def accumulate(buf, g):
        def rbody(r, accs):
            return tuple(
                accs[j] + buf[r, pl.ds(j * LANES, LANES)] for j in range(NV)
            )
        init = tuple(buf[0, pl.ds(j * LANES, LANES)] for j in range(NV))
        accs = lax.fori_loop(1, L, rbody, init, unroll=5)
        for j in range(NV):
            outblk[g, pl.ds(j * LANES, LANES)] = accs[j] * INV_L

    def block_body(blk):
        @pl.when(blk > 0)
        def _():
            pltpu.make_async_copy(
                outblk, out_hbm.at[pl.ds(base + (blk - 1) * IB, IB)], sob).wait()

        row0 = base + blk * IB
        pltpu.sync_copy(idx_hbm.at[pl.ds(row0, IB)], idx_v)
        lax.fori_loop(0, IB, remap, 0, unroll=2)
        for s in range(NBUF):
            pltpu.async_copy(
                table_hbm.at[idx_v.at[s]], rows[s], sems[s])

        def quad(p):
            for s in range(NBUF):
                g = NBUF * p + s
                pltpu.make_async_copy(
                    table_hbm.at[idx_v.at[g]], rows[s], sems[s]).wait()
                accumulate(rows[s], g)

                @pl.when(g + NBUF < IB)
                def _():
                    pltpu.async_copy(
                        table_hbm.at[idx_v.at[g + NBUF]], rows[s], sems[s])

        pl.loop(0, IB // NBUF)(quad)
        pltpu.async_copy(outblk, out_hbm.at[pl.ds(row0, IB)], sob)

    pl.loop(0, NB)(block_body)
    pltpu.make_async_copy(
        outblk, out_hbm.at[pl.ds(base + (NB - 1) * IB, IB)], sob).wait()


@jax.jit
def _pooled_lookup(idx, table_lin):
    mesh = plsc.VectorSubcoreMesh(core_axis_name="c", subcore_axis_name="s")
    return pl.kernel(
        _pool_body,
        out_type=jax.ShapeDtypeStruct((NG, D), jnp.float32),
        mesh=mesh,
        scratch_types=[
            pltpu.VMEM((IB, LG), jnp.int32),
            pltpu.VMEM((LG, D), jnp.float32),
            pltpu.VMEM((LG, D), jnp.float32),
            pltpu.VMEM((LG, D), jnp.float32),
            pltpu.VMEM((LG, D), jnp.float32),
            pltpu.VMEM((IB, D), jnp.float32),
            pltpu.SemaphoreType.DMA,
            pltpu.SemaphoreType.DMA,
            pltpu.SemaphoreType.DMA,
            pltpu.SemaphoreType.DMA,
            pltpu.SemaphoreType.DMA,
        ],
        compiler_params=pltpu.CompilerParams(use_tc_tiling_on_sc=False),
    )(idx, table_lin)


def _mm_body(x_ref, w_ref, b_ref, o_ref):
    o_ref[...] = (
        jnp.dot(x_ref[...], w_ref[...], preferred_element_type=jnp.float32)
        + b_ref[...]
    )


@jax.jit
def _project(pooled, W, b):
    blk = 4096
    return pl.pallas_call(
        _mm_body,
        grid=(NG // blk,),
        in_specs=[
            pl.BlockSpec((blk, D), lambda i: (i, 0)),
            pl.BlockSpec((D, D), lambda i: (0, 0)),
            pl.BlockSpec((1, D), lambda i: (0, 0)),
        ],
        out_specs=pl.BlockSpec((blk, D), lambda i: (i, 0)),
        out_shape=jax.ShapeDtypeStruct((NG, D), jnp.float32),
    )(pooled, W, b.reshape(1, D))


def kernel(query, candidate, table, W, b):
    idx = jnp.concatenate([query, candidate], axis=0).astype(jnp.int32)
    idx = jnp.concatenate([idx, idx[:, : LG - L]], axis=1)
    table2 = _relayout(table.T)
    table_lin = table2.reshape(VP, D)
    pooled = _pooled_lookup(idx, table_lin)
    out = _project(pooled, W, b)
    return (out[:B], out[B:])


# speedup vs baseline: 5.4302x; 1.0080x over previous
import jax
import jax.numpy as jnp
from jax import lax
from jax.experimental import pallas as pl
from jax.experimental.pallas import tpu as pltpu
from jax.experimental.pallas import tpu_sc as plsc

B = 16384
L = 50
D = 64
V = 1000000
NG = 2 * B
NW = 32
GPW = NG // NW
IB = 64
NB = GPW // IB
NBUF = 4
LANES = 16
NV = D // LANES
INV_L = 1.0 / L
LG = 56

RW = 2048
RH = RW // 2
NRB = (V + RW - 1) // RW
VP = NRB * RW


def _relayout_body(x_ref, o_ref):
    x = x_ref[...]
    eye = jnp.eye(D, dtype=jnp.float32)
    dn = (((0,), (0,)), ((), ()))
    lo = lax.dot_general(x[:, 0:RH], eye, dn, preferred_element_type=jnp.float32)
    hi = lax.dot_general(x[:, RH:RW], eye, dn, preferred_element_type=jnp.float32)
    o_ref[:, 0:D] = lo
    o_ref[:, D:2 * D] = hi


@jax.jit
def _relayout(tt):
    return pl.pallas_call(
        _relayout_body,
        grid=(NRB,),
        in_specs=[pl.BlockSpec((D, RW), lambda j: (0, j))],
        out_specs=pl.BlockSpec((RH, 2 * D), lambda j: (j, 0)),
        out_shape=jax.ShapeDtypeStruct((VP // 2, 2 * D), jnp.float32),
    )(tt)


def _pool_body(idx_hbm, table_hbm, out_hbm,
               idx_v, r0, r1, r2, r3, outblk, s0, s1, s2, s3, sob):
    wid = lax.axis_index("s") * 2 + lax.axis_index("c")
    base = wid * GPW
    rows = (r0, r1, r2, r3)
    sems = (s0, s1, s2, s3)

    def remap(r, _):
        vs = [idx_v[r, pl.ds(c, LANES)] for c in (0, 16, 32, 40)]
        outs = []
        for v in vs:
            off = v & (RW - 1)
            outs.append(v + off - jnp.where(off < RH, 0, RW - 1))
        for c, o in zip((0, 16, 32, 40), outs):
            idx_v[r, pl.ds(c, LANES)] = o
        return 0

    def accumulate(buf, g):
        def rbody(r, accs):
            return tuple(
                accs[j] + buf[r, pl.ds(j * LANES, LANES)] for j in range(NV)
            )
        init = tuple(buf[0, pl.ds(j * LANES, LANES)] for j in range(NV))
        accs = lax.fori_loop(1, L, rbody, init, unroll=5)
        for j in range(NV):
            outblk[g, pl.ds(j * LANES, LANES)] = accs[j] * INV_L

    def block_body(blk):
        @pl.when(blk > 0)
        def _():
            pltpu.make_async_copy(
                outblk, out_hbm.at[pl.ds(base + (blk - 1) * IB, IB)], sob).wait()

        row0 = base + blk * IB
        pltpu.sync_copy(idx_hbm.at[pl.ds(row0, IB)], idx_v)
        lax.fori_loop(0, IB, remap, 0, unroll=2)
        for s in range(NBUF):
            pltpu.async_copy(
                table_hbm.at[idx_v.at[s]], rows[s], sems[s])

        def quad(p):
            for s in range(NBUF):
                g = NBUF * p + s
                pltpu.make_async_copy(
                    table_hbm.at[idx_v.at[g]], rows[s], sems[s]).wait()
                accumulate(rows[s], g)

                @pl.when(g + NBUF < IB)
                def _():
                    pltpu.async_copy(
                        table_hbm.at[idx_v.at[g + NBUF]], rows[s], sems[s])

        pl.loop(0, IB // NBUF)(quad)
        pltpu.async_copy(outblk, out_hbm.at[pl.ds(row0, IB)], sob)

    pl.loop(0, NB)(block_body)
    pltpu.make_async_copy(
        outblk, out_hbm.at[pl.ds(base + (NB - 1) * IB, IB)], sob).wait()


@jax.jit
def _pooled_lookup(idx, table_lin):
    mesh = plsc.VectorSubcoreMesh(core_axis_name="c", subcore_axis_name="s")
    return pl.kernel(
        _pool_body,
        out_type=jax.ShapeDtypeStruct((NG, D), jnp.float32),
        mesh=mesh,
        scratch_types=[
            pltpu.VMEM((IB, LG), jnp.int32),
            pltpu.VMEM((LG, D), jnp.float32),
            pltpu.VMEM((LG, D), jnp.float32),
            pltpu.VMEM((LG, D), jnp.float32),
            pltpu.VMEM((LG, D), jnp.float32),
            pltpu.VMEM((IB, D), jnp.float32),
            pltpu.SemaphoreType.DMA,
            pltpu.SemaphoreType.DMA,
            pltpu.SemaphoreType.DMA,
            pltpu.SemaphoreType.DMA,
            pltpu.SemaphoreType.DMA,
        ],
        compiler_params=pltpu.CompilerParams(use_tc_tiling_on_sc=False),
    )(idx, table_lin)


def _mm_body(x_ref, w_ref, b_ref, o_ref):
    o_ref[...] = (
        jnp.dot(x_ref[...], w_ref[...], preferred_element_type=jnp.float32)
        + b_ref[...]
    )


@jax.jit
def _project(pooled, W, b):
    blk = 4096
    return pl.pallas_call(
        _mm_body,
        grid=(NG // blk,),
        in_specs=[
            pl.BlockSpec((blk, D), lambda i: (i, 0)),
            pl.BlockSpec((D, D), lambda i: (0, 0)),
            pl.BlockSpec((1, D), lambda i: (0, 0)),
        ],
        out_specs=pl.BlockSpec((blk, D), lambda i: (i, 0)),
        out_shape=jax.ShapeDtypeStruct((NG, D), jnp.float32),
    )(pooled, W, b.reshape(1, D))


def kernel(query, candidate, table, W, b):
    idx = jnp.concatenate([query, candidate], axis=0).astype(jnp.int32)
    idx = jnp.concatenate([idx, idx[:, : LG - L]], axis=1)
    table2 = _relayout(table.T)
    table_lin = table2.reshape(VP, D)
    pooled = _pooled_lookup(idx, table_lin)
    out = _project(pooled, W, b)
    return (out[:B], out[B:])


# speedup vs baseline: 6.4230x; 1.1828x over previous
import jax
import jax.numpy as jnp
from jax import lax
from jax.experimental import pallas as pl
from jax.experimental.pallas import tpu as pltpu
from jax.experimental.pallas import tpu_sc as plsc

B = 16384
L = 50
D = 64
V = 1000000
NG = 2 * B
NW = 32
GPW = NG // NW
IB = 64
NB = GPW // IB
NBUF = 4
LANES = 16
NV = D // LANES
INV_L = 1.0 / L
LG = 56

RW = 4096
RH = RW // 2
NRB = (V + RW - 1) // RW
VP = NRB * RW


def _relayout_body(x_ref, o_ref):
    x = x_ref[...]
    eye = jnp.eye(D, dtype=jnp.float32)
    dn = (((0,), (0,)), ((), ()))
    lo = lax.dot_general(x[:, 0:RH], eye, dn, preferred_element_type=jnp.float32)
    hi = lax.dot_general(x[:, RH:RW], eye, dn, preferred_element_type=jnp.float32)
    o_ref[:, 0:D] = lo
    o_ref[:, D:2 * D] = hi


@jax.jit
def _relayout(tt):
    return pl.pallas_call(
        _relayout_body,
        grid=(NRB,),
        in_specs=[pl.BlockSpec((D, RW), lambda j: (0, j))],
        out_specs=pl.BlockSpec((RH, 2 * D), lambda j: (j, 0)),
        out_shape=jax.ShapeDtypeStruct((VP // 2, 2 * D), jnp.float32),
    )(tt)


def _pool_body(idx_hbm, table_hbm, out_hbm,
               idx_v, r0, r1, r2, r3, outblk, s0, s1, s2, s3, sob):
    wid = lax.axis_index("s") * 2 + lax.axis_index("c")
    base = wid * GPW
    rows = (r0, r1, r2, r3)
    sems = (s0, s1, s2, s3)

    def remap(r, _):
        vs = [idx_v[r, pl.ds(c, LANES)] for c in (0, 16, 32, 40)]
        outs = []
        for v in vs:
            off = v & (RW - 1)
            outs.append(v + off - jnp.where(off < RH, 0, RW - 1))
        for c, o in zip((0, 16, 32, 40), outs):
            idx_v[r, pl.ds(c, LANES)] = o
        return 0

    def accumulate(buf, g):
        def rbody(r, accs):
            return tuple(
                accs[j] + buf[r, pl.ds(j * LANES, LANES)] for j in range(NV)
            )
        init = tuple(buf[0, pl.ds(j * LANES, LANES)] for j in range(NV))
        accs = lax.fori_loop(1, L, rbody, init, unroll=5)
        for j in range(NV):
            outblk[g, pl.ds(j * LANES, LANES)] = accs[j] * INV_L

    def block_body(blk):
        @pl.when(blk > 0)
        def _():
            pltpu.make_async_copy(
                outblk, out_hbm.at[pl.ds(base + (blk - 1) * IB, IB)], sob).wait()

        row0 = base + blk * IB
        pltpu.sync_copy(idx_hbm.at[pl.ds(row0, IB)], idx_v)
        lax.fori_loop(0, IB, remap, 0, unroll=2)
        for s in range(NBUF):
            pltpu.async_copy(
                table_hbm.at[idx_v.at[s]], rows[s], sems[s])

        def quad(p):
            for s in range(NBUF):
                g = NBUF * p + s
                pltpu.make_async_copy(
                    table_hbm.at[idx_v.at[g]], rows[s], sems[s]).wait()
                accumulate(rows[s], g)

                @pl.when(g + NBUF < IB)
                def _():
                    pltpu.async_copy(
                        table_hbm.at[idx_v.at[g + NBUF]], rows[s], sems[s])

        pl.loop(0, IB // NBUF)(quad)
        pltpu.async_copy(outblk, out_hbm.at[pl.ds(row0, IB)], sob)

    pl.loop(0, NB)(block_body)
    pltpu.make_async_copy(
        outblk, out_hbm.at[pl.ds(base + (NB - 1) * IB, IB)], sob).wait()


@jax.jit
def _pooled_lookup(idx, table_lin):
    mesh = plsc.VectorSubcoreMesh(core_axis_name="c", subcore_axis_name="s")
    return pl.kernel(
        _pool_body,
        out_type=jax.ShapeDtypeStruct((NG, D), jnp.float32),
        mesh=mesh,
        scratch_types=[
            pltpu.VMEM((IB, LG), jnp.int32),
            pltpu.VMEM((LG, D), jnp.float32),
            pltpu.VMEM((LG, D), jnp.float32),
            pltpu.VMEM((LG, D), jnp.float32),
            pltpu.VMEM((LG, D), jnp.float32),
            pltpu.VMEM((IB, D), jnp.float32),
            pltpu.SemaphoreType.DMA,
            pltpu.SemaphoreType.DMA,
            pltpu.SemaphoreType.DMA,
            pltpu.SemaphoreType.DMA,
            pltpu.SemaphoreType.DMA,
        ],
        compiler_params=pltpu.CompilerParams(use_tc_tiling_on_sc=False),
    )(idx, table_lin)


def _mm_body(x_ref, w_ref, b_ref, o_ref):
    o_ref[...] = (
        jnp.dot(x_ref[...], w_ref[...], preferred_element_type=jnp.float32)
        + b_ref[...]
    )


@jax.jit
def _project(pooled, W, b):
    blk = 4096
    return pl.pallas_call(
        _mm_body,
        grid=(NG // blk,),
        in_specs=[
            pl.BlockSpec((blk, D), lambda i: (i, 0)),
            pl.BlockSpec((D, D), lambda i: (0, 0)),
            pl.BlockSpec((1, D), lambda i: (0, 0)),
        ],
        out_specs=pl.BlockSpec((blk, D), lambda i: (i, 0)),
        out_shape=jax.ShapeDtypeStruct((NG, D), jnp.float32),
    )(pooled, W, b.reshape(1, D))


def kernel(query, candidate, table, W, b):
    idx = jnp.concatenate([query, candidate], axis=0).astype(jnp.int32)
    idx = jnp.concatenate([idx, idx[:, : LG - L]], axis=1)
    table2 = _relayout(table.T)
    table_lin = table2.reshape(VP, D)
    pooled = _pooled_lookup(idx, table_lin)
    out = _project(pooled, W, b)
    return (out[:B], out[B:])


# speedup vs baseline: 7.1013x; 1.1056x over previous
import jax
import jax.numpy as jnp
from jax import lax
from jax.experimental import pallas as pl
from jax.experimental.pallas import tpu as pltpu
from jax.experimental.pallas import tpu_sc as plsc

B = 16384
L = 50
D = 64
V = 1000000
NG = 2 * B
NW = 32
GPW = NG // NW
IB = 64
NB = GPW // IB
NBUF = 4
LANES = 16
NV = D // LANES
INV_L = 1.0 / L
LG = 56

RW = 8192
RH = RW // 2
NRB = (V + RW - 1) // RW
VP = NRB * RW


def _relayout_body(x_ref, o_ref):
    x = x_ref[...]
    eye = jnp.eye(D, dtype=jnp.float32)
    dn = (((0,), (0,)), ((), ()))
    lo = lax.dot_general(x[:, 0:RH], eye, dn, preferred_element_type=jnp.float32)
    hi = lax.dot_general(x[:, RH:RW], eye, dn, preferred_element_type=jnp.float32)
    o_ref[:, 0:D] = lo
    o_ref[:, D:2 * D] = hi


@jax.jit
def _relayout(tt):
    return pl.pallas_call(
        _relayout_body,
        grid=(NRB,),
        in_specs=[pl.BlockSpec((D, RW), lambda j: (0, j))],
        out_specs=pl.BlockSpec((RH, 2 * D), lambda j: (j, 0)),
        out_shape=jax.ShapeDtypeStruct((VP // 2, 2 * D), jnp.float32),
    )(tt)


def _pool_body(idx_hbm, table_hbm, out_hbm,
               idx_v, r0, r1, r2, r3, outblk, s0, s1, s2, s3, sob):
    wid = lax.axis_index("s") * 2 + lax.axis_index("c")
    base = wid * GPW
    rows = (r0, r1, r2, r3)
    sems = (s0, s1, s2, s3)

    def remap(r, _):
        vs = [idx_v[r, pl.ds(c, LANES)] for c in (0, 16, 32, 40)]
        outs = []
        for v in vs:
            off = v & (RW - 1)
            outs.append(v + off - jnp.where(off < RH, 0, RW - 1))
        for c, o in zip((0, 16, 32, 40), outs):
            idx_v[r, pl.ds(c, LANES)] = o
        return 0

    def accumulate(buf, g):
        def rbody(r, accs):
            return tuple(
                accs[j] + buf[r, pl.ds(j * LANES, LANES)] for j in range(NV)
            )
        init = tuple(buf[0, pl.ds(j * LANES, LANES)] for j in range(NV))
        accs = lax.fori_loop(1, L, rbody, init, unroll=5)
        for j in range(NV):
            outblk[g, pl.ds(j * LANES, LANES)] = accs[j] * INV_L

    def block_body(blk):
        @pl.when(blk > 0)
        def _():
            pltpu.make_async_copy(
                outblk, out_hbm.at[pl.ds(base + (blk - 1) * IB, IB)], sob).wait()

        row0 = base + blk * IB
        pltpu.sync_copy(idx_hbm.at[pl.ds(row0, IB)], idx_v)
        lax.fori_loop(0, IB, remap, 0, unroll=2)
        for s in range(NBUF):
            pltpu.async_copy(
                table_hbm.at[idx_v.at[s]], rows[s], sems[s])

        def quad(p):
            for s in range(NBUF):
                g = NBUF * p + s
                pltpu.make_async_copy(
                    table_hbm.at[idx_v.at[g]], rows[s], sems[s]).wait()
                accumulate(rows[s], g)

                @pl.when(g + NBUF < IB)
                def _():
                    pltpu.async_copy(
                        table_hbm.at[idx_v.at[g + NBUF]], rows[s], sems[s])

        pl.loop(0, IB // NBUF)(quad)
        pltpu.async_copy(outblk, out_hbm.at[pl.ds(row0, IB)], sob)

    pl.loop(0, NB)(block_body)
    pltpu.make_async_copy(
        outblk, out_hbm.at[pl.ds(base + (NB - 1) * IB, IB)], sob).wait()


@jax.jit
def _pooled_lookup(idx, table_lin):
    mesh = plsc.VectorSubcoreMesh(core_axis_name="c", subcore_axis_name="s")
    return pl.kernel(
        _pool_body,
        out_type=jax.ShapeDtypeStruct((NG, D), jnp.float32),
        mesh=mesh,
        scratch_types=[
            pltpu.VMEM((IB, LG), jnp.int32),
            pltpu.VMEM((LG, D), jnp.float32),
            pltpu.VMEM((LG, D), jnp.float32),
            pltpu.VMEM((LG, D), jnp.float32),
            pltpu.VMEM((LG, D), jnp.float32),
            pltpu.VMEM((IB, D), jnp.float32),
            pltpu.SemaphoreType.DMA,
            pltpu.SemaphoreType.DMA,
            pltpu.SemaphoreType.DMA,
            pltpu.SemaphoreType.DMA,
            pltpu.SemaphoreType.DMA,
        ],
        compiler_params=pltpu.CompilerParams(use_tc_tiling_on_sc=False),
    )(idx, table_lin)


def _mm_body(x_ref, w_ref, b_ref, o_ref):
    o_ref[...] = (
        jnp.dot(x_ref[...], w_ref[...], preferred_element_type=jnp.float32)
        + b_ref[...]
    )


@jax.jit
def _project(pooled, W, b):
    blk = 4096
    return pl.pallas_call(
        _mm_body,
        grid=(NG // blk,),
        in_specs=[
            pl.BlockSpec((blk, D), lambda i: (i, 0)),
            pl.BlockSpec((D, D), lambda i: (0, 0)),
            pl.BlockSpec((1, D), lambda i: (0, 0)),
        ],
        out_specs=pl.BlockSpec((blk, D), lambda i: (i, 0)),
        out_shape=jax.ShapeDtypeStruct((NG, D), jnp.float32),
    )(pooled, W, b.reshape(1, D))


def kernel(query, candidate, table, W, b):
    idx = jnp.concatenate([query, candidate], axis=0).astype(jnp.int32)
    idx = jnp.concatenate([idx, idx[:, : LG - L]], axis=1)
    table2 = _relayout(table.T)
    table_lin = table2.reshape(VP, D)
    pooled = _pooled_lookup(idx, table_lin)
    out = _project(pooled, W, b)
    return (out[:B], out[B:])


# speedup vs baseline: 7.4958x; 1.0555x over previous
import jax
import jax.numpy as jnp
from jax import lax
from jax.experimental import pallas as pl
from jax.experimental.pallas import tpu as pltpu
from jax.experimental.pallas import tpu_sc as plsc

B = 16384
L = 50
D = 64
V = 1000000
NG = 2 * B
NW = 32
GPW = NG // NW
IB = 64
NB = GPW // IB
NBUF = 4
LANES = 16
NV = D // LANES
INV_L = 1.0 / L
LG = 56

RW = 16384
RH = RW // 2
NRB = (V + RW - 1) // RW
VP = NRB * RW


def _relayout_body(x_ref, o_ref):
    x = x_ref[...]
    eye = jnp.eye(D, dtype=jnp.float32)
    dn = (((0,), (0,)), ((), ()))
    lo = lax.dot_general(x[:, 0:RH], eye, dn, preferred_element_type=jnp.float32)
    hi = lax.dot_general(x[:, RH:RW], eye, dn, preferred_element_type=jnp.float32)
    o_ref[:, 0:D] = lo
    o_ref[:, D:2 * D] = hi


@jax.jit
def _relayout(tt):
    return pl.pallas_call(
        _relayout_body,
        grid=(NRB,),
        in_specs=[pl.BlockSpec((D, RW), lambda j: (0, j))],
        out_specs=pl.BlockSpec((RH, 2 * D), lambda j: (j, 0)),
        out_shape=jax.ShapeDtypeStruct((VP // 2, 2 * D), jnp.float32),
    )(tt)


def _pool_body(idx_hbm, table_hbm, out_hbm,
               idx_v, r0, r1, r2, r3, outblk, s0, s1, s2, s3, sob):
    wid = lax.axis_index("s") * 2 + lax.axis_index("c")
    base = wid * GPW
    rows = (r0, r1, r2, r3)
    sems = (s0, s1, s2, s3)

    def remap(r, _):
        vs = [idx_v[r, pl.ds(c, LANES)] for c in (0, 16, 32, 40)]
        outs = []
        for v in vs:
            off = v & (RW - 1)
            outs.append(v + off - jnp.where(off < RH, 0, RW - 1))
        for c, o in zip((0, 16, 32, 40), outs):
            idx_v[r, pl.ds(c, LANES)] = o
        return 0

    def accumulate(buf, g):
        def rbody(r, accs):
            return tuple(
                accs[j] + buf[r, pl.ds(j * LANES, LANES)] for j in range(NV)
            )
        init = tuple(buf[0, pl.ds(j * LANES, LANES)] for j in range(NV))
        accs = lax.fori_loop(1, L, rbody, init, unroll=5)
        for j in range(NV):
            outblk[g, pl.ds(j * LANES, LANES)] = accs[j] * INV_L

    def block_body(blk):
        @pl.when(blk > 0)
        def _():
            pltpu.make_async_copy(
                outblk, out_hbm.at[pl.ds(base + (blk - 1) * IB, IB)], sob).wait()

        row0 = base + blk * IB
        pltpu.sync_copy(idx_hbm.at[pl.ds(row0, IB)], idx_v)
        lax.fori_loop(0, IB, remap, 0, unroll=2)
        for s in range(NBUF):
            pltpu.async_copy(
                table_hbm.at[idx_v.at[s]], rows[s], sems[s])

        def quad(p):
            for s in range(NBUF):
                g = NBUF * p + s
                pltpu.make_async_copy(
                    table_hbm.at[idx_v.at[g]], rows[s], sems[s]).wait()
                accumulate(rows[s], g)

                @pl.when(g + NBUF < IB)
                def _():
                    pltpu.async_copy(
                        table_hbm.at[idx_v.at[g + NBUF]], rows[s], sems[s])

        pl.loop(0, IB // NBUF)(quad)
        pltpu.async_copy(outblk, out_hbm.at[pl.ds(row0, IB)], sob)

    pl.loop(0, NB)(block_body)
    pltpu.make_async_copy(
        outblk, out_hbm.at[pl.ds(base + (NB - 1) * IB, IB)], sob).wait()


@jax.jit
def _pooled_lookup(idx, table_lin):
    mesh = plsc.VectorSubcoreMesh(core_axis_name="c", subcore_axis_name="s")
    return pl.kernel(
        _pool_body,
        out_type=jax.ShapeDtypeStruct((NG, D), jnp.float32),
        mesh=mesh,
        scratch_types=[
            pltpu.VMEM((IB, LG), jnp.int32),
            pltpu.VMEM((LG, D), jnp.float32),
            pltpu.VMEM((LG, D), jnp.float32),
            pltpu.VMEM((LG, D), jnp.float32),
            pltpu.VMEM((LG, D), jnp.float32),
            pltpu.VMEM((IB, D), jnp.float32),
            pltpu.SemaphoreType.DMA,
            pltpu.SemaphoreType.DMA,
            pltpu.SemaphoreType.DMA,
            pltpu.SemaphoreType.DMA,
            pltpu.SemaphoreType.DMA,
        ],
        compiler_params=pltpu.CompilerParams(use_tc_tiling_on_sc=False),
    )(idx, table_lin)


def _mm_body(x_ref, w_ref, b_ref, o_ref):
    o_ref[...] = (
        jnp.dot(x_ref[...], w_ref[...], preferred_element_type=jnp.float32)
        + b_ref[...]
    )


@jax.jit
def _project(pooled, W, b):
    blk = 4096
    return pl.pallas_call(
        _mm_body,
        grid=(NG // blk,),
        in_specs=[
            pl.BlockSpec((blk, D), lambda i: (i, 0)),
            pl.BlockSpec((D, D), lambda i: (0, 0)),
            pl.BlockSpec((1, D), lambda i: (0, 0)),
        ],
        out_specs=pl.BlockSpec((blk, D), lambda i: (i, 0)),
        out_shape=jax.ShapeDtypeStruct((NG, D), jnp.float32),
    )(pooled, W, b.reshape(1, D))


def kernel(query, candidate, table, W, b):
    idx = jnp.concatenate([query, candidate], axis=0).astype(jnp.int32)
    idx = jnp.concatenate([idx, idx[:, : LG - L]], axis=1)
    table2 = _relayout(table.T)
    table_lin = table2.reshape(VP, D)
    pooled = _pooled_lookup(idx, table_lin)
    out = _project(pooled, W, b)
    return (out[:B], out[B:])


# speedup vs baseline: 7.7064x; 1.0281x over previous
import jax
import jax.numpy as jnp
from jax import lax
from jax.experimental import pallas as pl
from jax.experimental.pallas import tpu as pltpu
from jax.experimental.pallas import tpu_sc as plsc

B = 16384
L = 50
D = 64
V = 1000000
NG = 2 * B
NW = 32
GPW = NG // NW
IB = 64
NB = GPW // IB
NBUF = 4
LANES = 16
NV = D // LANES
INV_L = 1.0 / L
LG = 56

RW = 32768
RH = RW // 2
NRB = (V + RW - 1) // RW
VP = NRB * RW


def _relayout_body(x_ref, o_ref):
    x = x_ref[...]
    eye = jnp.eye(D, dtype=jnp.float32)
    dn = (((0,), (0,)), ((), ()))
    lo = lax.dot_general(x[:, 0:RH], eye, dn, preferred_element_type=jnp.float32)
    hi = lax.dot_general(x[:, RH:RW], eye, dn, preferred_element_type=jnp.float32)
    o_ref[:, 0:D] = lo
    o_ref[:, D:2 * D] = hi


@jax.jit
def _relayout(tt):
    return pl.pallas_call(
        _relayout_body,
        grid=(NRB,),
        in_specs=[pl.BlockSpec((D, RW), lambda j: (0, j))],
        out_specs=pl.BlockSpec((RH, 2 * D), lambda j: (j, 0)),
        out_shape=jax.ShapeDtypeStruct((VP // 2, 2 * D), jnp.float32),
    )(tt)


def _pool_body(idx_hbm, table_hbm, out_hbm,
               idx_v, r0, r1, r2, r3, outblk, s0, s1, s2, s3, sob):
    wid = lax.axis_index("s") * 2 + lax.axis_index("c")
    base = wid * GPW
    rows = (r0, r1, r2, r3)
    sems = (s0, s1, s2, s3)

    def remap(r, _):
        vs = [idx_v[r, pl.ds(c, LANES)] for c in (0, 16, 32, 40)]
        outs = []
        for v in vs:
            off = v & (RW - 1)
            outs.append(v + off - jnp.where(off < RH, 0, RW - 1))
        for c, o in zip((0, 16, 32, 40), outs):
            idx_v[r, pl.ds(c, LANES)] = o
        return 0

    def accumulate(buf, g):
        def rbody(r, accs):
            return tuple(
                accs[j] + buf[r, pl.ds(j * LANES, LANES)] for j in range(NV)
            )
        init = tuple(buf[0, pl.ds(j * LANES, LANES)] for j in range(NV))
        accs = lax.fori_loop(1, L, rbody, init, unroll=5)
        for j in range(NV):
            outblk[g, pl.ds(j * LANES, LANES)] = accs[j] * INV_L

    def block_body(blk):
        @pl.when(blk > 0)
        def _():
            pltpu.make_async_copy(
                outblk, out_hbm.at[pl.ds(base + (blk - 1) * IB, IB)], sob).wait()

        row0 = base + blk * IB
        pltpu.sync_copy(idx_hbm.at[pl.ds(row0, IB)], idx_v)
        lax.fori_loop(0, IB, remap, 0, unroll=2)
        for s in range(NBUF):
            pltpu.async_copy(
                table_hbm.at[idx_v.at[s]], rows[s], sems[s])

        def quad(p):
            for s in range(NBUF):
                g = NBUF * p + s
                pltpu.make_async_copy(
                    table_hbm.at[idx_v.at[g]], rows[s], sems[s]).wait()
                accumulate(rows[s], g)

                @pl.when(g + NBUF < IB)
                def _():
                    pltpu.async_copy(
                        table_hbm.at[idx_v.at[g + NBUF]], rows[s], sems[s])

        pl.loop(0, IB // NBUF)(quad)
        pltpu.async_copy(outblk, out_hbm.at[pl.ds(row0, IB)], sob)

    pl.loop(0, NB)(block_body)
    pltpu.make_async_copy(
        outblk, out_hbm.at[pl.ds(base + (NB - 1) * IB, IB)], sob).wait()


@jax.jit
def _pooled_lookup(idx, table_lin):
    mesh = plsc.VectorSubcoreMesh(core_axis_name="c", subcore_axis_name="s")
    return pl.kernel(
        _pool_body,
        out_type=jax.ShapeDtypeStruct((NG, D), jnp.float32),
        mesh=mesh,
        scratch_types=[
            pltpu.VMEM((IB, LG), jnp.int32),
            pltpu.VMEM((LG, D), jnp.float32),
            pltpu.VMEM((LG, D), jnp.float32),
            pltpu.VMEM((LG, D), jnp.float32),
            pltpu.VMEM((LG, D), jnp.float32),
            pltpu.VMEM((IB, D), jnp.float32),
            pltpu.SemaphoreType.DMA,
            pltpu.SemaphoreType.DMA,
            pltpu.SemaphoreType.DMA,
            pltpu.SemaphoreType.DMA,
            pltpu.SemaphoreType.DMA,
        ],
        compiler_params=pltpu.CompilerParams(use_tc_tiling_on_sc=False),
    )(idx, table_lin)


def _mm_body(x_ref, w_ref, b_ref, o_ref):
    o_ref[...] = (
        jnp.dot(x_ref[...], w_ref[...], preferred_element_type=jnp.float32)
        + b_ref[...]
    )


@jax.jit
def _project(pooled, W, b):
    blk = 4096
    return pl.pallas_call(
        _mm_body,
        grid=(NG // blk,),
        in_specs=[
            pl.BlockSpec((blk, D), lambda i: (i, 0)),
            pl.BlockSpec((D, D), lambda i: (0, 0)),
            pl.BlockSpec((1, D), lambda i: (0, 0)),
        ],
        out_specs=pl.BlockSpec((blk, D), lambda i: (i, 0)),
        out_shape=jax.ShapeDtypeStruct((NG, D), jnp.float32),
    )(pooled, W, b.reshape(1, D))


def kernel(query, candidate, table, W, b):
    idx = jnp.concatenate([query, candidate], axis=0).astype(jnp.int32)
    idx = jnp.concatenate([idx, idx[:, : LG - L]], axis=1)
    table2 = _relayout(table.T)
    table_lin = table2.reshape(VP, D)
    pooled = _pooled_lookup(idx, table_lin)
    out = _project(pooled, W, b)
    return (out[:B], out[B:])


# speedup vs baseline: 7.8592x; 1.0198x over previous
import jax
import jax.numpy as jnp
from jax import lax
from jax.experimental import pallas as pl
from jax.experimental.pallas import tpu as pltpu
from jax.experimental.pallas import tpu_sc as plsc

B = 16384
L = 50
D = 64
V = 1000000
NG = 2 * B
NW = 32
GPW = NG // NW
IB = 64
NB = GPW // IB
NBUF = 4
LANES = 16
NV = D // LANES
INV_L = 1.0 / L
LG = 56

RW = 32768
RH = RW // 2
NRB = (V + RW - 1) // RW
VP = NRB * RW


def _relayout_body(x_ref, o_ref):
    x = x_ref[...]
    eye = jnp.eye(D, dtype=jnp.float32)
    dn = (((0,), (0,)), ((), ()))
    lo = lax.dot_general(x[:, 0:RH], eye, dn, preferred_element_type=jnp.float32)
    hi = lax.dot_general(x[:, RH:RW], eye, dn, preferred_element_type=jnp.float32)
    o_ref[:, 0:D] = lo
    o_ref[:, D:2 * D] = hi


@jax.jit
def _relayout(tt):
    return pl.pallas_call(
        _relayout_body,
        grid=(NRB,),
        in_specs=[pl.BlockSpec((D, RW), lambda j: (0, j))],
        out_specs=pl.BlockSpec((RH, 2 * D), lambda j: (j, 0)),
        out_shape=jax.ShapeDtypeStruct((VP // 2, 2 * D), jnp.float32),
    )(tt)


def _pool_body(idx_hbm, table_hbm, out_hbm,
               idx_v, r0, r1, r2, r3, outblk, s0, s1, s2, s3, sob):
    wid = lax.axis_index("s") * 2 + lax.axis_index("c")
    base = wid * GPW
    rows = (r0, r1, r2, r3)
    sems = (s0, s1, s2, s3)

    def remap(r, _):
        vs = [idx_v[r, pl.ds(c, LANES)] for c in (0, 16, 32, 40)]
        outs = []
        for v in vs:
            off = v & (RW - 1)
            outs.append(v + off - jnp.where(off < RH, 0, RW - 1))
        for c, o in zip((0, 16, 32, 40), outs):
            idx_v[r, pl.ds(c, LANES)] = o
        return 0

    def accumulate(buf, g):
        def rbody(r, accs):
            return tuple(
                accs[j] + buf[r, pl.ds(j * LANES, LANES)] for j in range(NV)
            )
        init = tuple(buf[0, pl.ds(j * LANES, LANES)] for j in range(NV))
        accs = lax.fori_loop(1, L, rbody, init, unroll=5)
        for j in range(NV):
            outblk[g, pl.ds(j * LANES, LANES)] = accs[j] * INV_L

    def block_body(blk):
        @pl.when(blk > 0)
        def _():
            pltpu.make_async_copy(
                outblk, out_hbm.at[pl.ds(base + (blk - 1) * IB, IB)], sob).wait()

        row0 = base + blk * IB
        pltpu.sync_copy(idx_hbm.at[pl.ds(row0, IB)], idx_v)
        lax.fori_loop(0, IB, remap, 0, unroll=2)
        for s in range(NBUF):
            pltpu.async_copy(
                table_hbm.at[idx_v.at[s]], rows[s], sems[s])

        def quad(p):
            for s in range(NBUF):
                g = NBUF * p + s
                pltpu.make_async_copy(
                    table_hbm.at[idx_v.at[g]], rows[s], sems[s]).wait()
                accumulate(rows[s], g)

                @pl.when(g + NBUF < IB)
                def _():
                    pltpu.async_copy(
                        table_hbm.at[idx_v.at[g + NBUF]], rows[s], sems[s])

        pl.loop(0, IB // NBUF)(quad)
        pltpu.async_copy(outblk, out_hbm.at[pl.ds(row0, IB)], sob)

    pl.loop(0, NB)(block_body)
    pltpu.make_async_copy(
        outblk, out_hbm.at[pl.ds(base + (NB - 1) * IB, IB)], sob).wait()


@jax.jit
def _pooled_lookup(idx, table_lin):
    mesh = plsc.VectorSubcoreMesh(core_axis_name="c", subcore_axis_name="s")
    return pl.kernel(
        _pool_body,
        out_type=jax.ShapeDtypeStruct((NG, D), jnp.float32),
        mesh=mesh,
        scratch_types=[
            pltpu.VMEM((IB, LG), jnp.int32),
            pltpu.VMEM((LG, D), jnp.float32),
            pltpu.VMEM((LG, D), jnp.float32),
            pltpu.VMEM((LG, D), jnp.float32),
            pltpu.VMEM((LG, D), jnp.float32),
            pltpu.VMEM((IB, D), jnp.float32),
            pltpu.SemaphoreType.DMA,
            pltpu.SemaphoreType.DMA,
            pltpu.SemaphoreType.DMA,
            pltpu.SemaphoreType.DMA,
            pltpu.SemaphoreType.DMA,
        ],
        compiler_params=pltpu.CompilerParams(use_tc_tiling_on_sc=False),
    )(idx, table_lin)


def _mm_body(q_ref, c_ref, w_ref, b_ref, qo_ref, co_ref):
    w = w_ref[...]
    bb = b_ref[...]
    qo_ref[...] = jnp.dot(q_ref[...], w, preferred_element_type=jnp.float32) + bb
    co_ref[...] = jnp.dot(c_ref[...], w, preferred_element_type=jnp.float32) + bb


@jax.jit
def _project(pooled, W, b):
    blk = 4096
    nblk = B // blk
    out_t = jax.ShapeDtypeStruct((B, D), jnp.float32)
    return pl.pallas_call(
        _mm_body,
        grid=(nblk,),
        in_specs=[
            pl.BlockSpec((blk, D), lambda i: (i, 0)),
            pl.BlockSpec((blk, D), lambda i: (i + B // blk, 0)),
            pl.BlockSpec((D, D), lambda i: (0, 0)),
            pl.BlockSpec((1, D), lambda i: (0, 0)),
        ],
        out_specs=[
            pl.BlockSpec((blk, D), lambda i: (i, 0)),
            pl.BlockSpec((blk, D), lambda i: (i, 0)),
        ],
        out_shape=(out_t, out_t),
    )(pooled, pooled, W, b.reshape(1, D))


def kernel(query, candidate, table, W, b):
    idx = jnp.concatenate([query, candidate], axis=0).astype(jnp.int32)
    idx = jnp.concatenate([idx, idx[:, : LG - L]], axis=1)
    table2 = _relayout(table.T)
    table_lin = table2.reshape(VP, D)
    pooled = _pooled_lookup(idx, table_lin)
    q_out, c_out = _project(pooled, W, b)
    return (q_out, c_out)
